# baseline (speedup 1.0000x reference)
import jax
import jax.numpy as jnp
from jax.experimental import pallas as pl

_MAX_STEPS = 4
_NUM_CLASSES = 2


def _mp_kernel(s_ref, x_ref, lab_ref, d_ref, ta_ref, tb_ref,
               recon_ref, pred_ref, sq_ref, ce_ref):
    i = pl.program_id(0)
    x = x_ref[...]
    D = d_ref[...]
    tA = ta_ref[...]
    tB = tb_ref[...]
    s = s_ref[0, 0]
    bt = x.shape[0]
    feat = x.shape[1]
    na = D.shape[1]

    residual = x
    recon = jnp.zeros_like(x)
    pred = jnp.zeros((bt, _NUM_CLASSES), jnp.float32)
    iota = jax.lax.broadcasted_iota(jnp.int32, (bt, na), 1)

    for t in range(_MAX_STEPS):
        proj = jnp.dot(residual, D, preferred_element_type=jnp.float32)
        absp = jnp.abs(proj)
        m = jnp.max(absp, axis=1, keepdims=True)
        best = jnp.min(jnp.where(absp == m, iota, na), axis=1, keepdims=True)
        hit = iota == best
        val = jnp.sum(jnp.where(hit, proj, 0.0), axis=1, keepdims=True)
        val = jnp.where(t < s, val, 0.0)
        dn = (((1,), (0,)), ((), ()))
        oh = hit.astype(jnp.bfloat16)
        dA = jax.lax.dot_general(oh, tA, dn,
                                 preferred_element_type=jnp.float32)
        dB = jax.lax.dot_general(oh, tB, dn,
                                 preferred_element_type=jnp.float32)
        atom = (dA[:, :feat] + dA[:, feat:2 * feat]) + dB[:, :feat]
        c01 = dB[:, feat:feat + 2 * _NUM_CLASSES]
        cvec = ((c01[:, :_NUM_CLASSES] + c01[:, _NUM_CLASSES:])
                + dB[:, feat + 2 * _NUM_CLASSES:feat + 3 * _NUM_CLASSES])
        upd = val * atom
        residual = residual - upd
        recon = recon + upd
        pred = pred + val * cvec

    recon_ref[...] = recon
    pred_ref[...] = pred

    lab = lab_ref[...]
    sq = jnp.sum((recon - x) ** 2).reshape(1, 1)
    mx = jnp.max(pred, axis=1, keepdims=True)
    lse = mx + jnp.log(jnp.sum(jnp.exp(pred - mx), axis=1, keepdims=True))
    ce = jnp.sum(lse[:, 0] - jnp.sum(pred * lab, axis=1)).reshape(1, 1)

    prev_sq = jnp.where(i == 0, jnp.zeros_like(sq), sq_ref[...])
    prev_ce = jnp.where(i == 0, jnp.zeros_like(ce), ce_ref[...])
    sq_ref[...] = prev_sq + sq
    ce_ref[...] = prev_ce + ce


def kernel(X, sparsity, labels, dictionary, classifier, beta):
    B, F = X.shape
    A = dictionary.shape[1]
    bt = 512
    nt = B // bt

    s = jnp.asarray(sparsity, jnp.int32).reshape(1, 1)
    lab1h = (labels[:, None] ==
             jnp.arange(_NUM_CLASSES, dtype=labels.dtype)[None, :]
             ).astype(jnp.float32)
    def split3(x):
        p0 = jax.lax.reduce_precision(x, 8, 7)
        r = x - p0
        p1 = jax.lax.reduce_precision(r, 8, 7)
        p2 = r - p1
        return (p0.astype(jnp.bfloat16), p1.astype(jnp.bfloat16),
                p2.astype(jnp.bfloat16))

    a0, a1, a2 = split3(dictionary.T)
    c0, c1, c2 = split3(classifier.T.astype(jnp.float32))
    tA = jnp.concatenate([a0, a1], axis=1)
    tB = jnp.concatenate([a2, c0, c1, c2], axis=1)

    recon, pred, sq, ce = pl.pallas_call(
        _mp_kernel,
        grid=(nt,),
        in_specs=[
            pl.BlockSpec((1, 1), lambda i: (0, 0)),
            pl.BlockSpec((bt, F), lambda i: (i, 0)),
            pl.BlockSpec((bt, _NUM_CLASSES), lambda i: (i, 0)),
            pl.BlockSpec((F, A), lambda i: (0, 0)),
            pl.BlockSpec((A, 2 * F), lambda i: (0, 0)),
            pl.BlockSpec((A, F + 3 * _NUM_CLASSES), lambda i: (0, 0)),
        ],
        out_specs=[
            pl.BlockSpec((bt, F), lambda i: (i, 0)),
            pl.BlockSpec((bt, _NUM_CLASSES), lambda i: (i, 0)),
            pl.BlockSpec((1, 1), lambda i: (0, 0)),
            pl.BlockSpec((1, 1), lambda i: (0, 0)),
        ],
        out_shape=[
            jax.ShapeDtypeStruct((B, F), jnp.float32),
            jax.ShapeDtypeStruct((B, _NUM_CLASSES), jnp.float32),
            jax.ShapeDtypeStruct((1, 1), jnp.float32),
            jax.ShapeDtypeStruct((1, 1), jnp.float32),
        ],
    )(s, X, lab1h, dictionary, tA, tB)

    mse = sq[0, 0] / (B * F)
    ce_mean = ce[0, 0] / B
    loss = (mse + beta * ce_mean).astype(jnp.float32)
    return recon, pred, loss


# speedup vs baseline: 1.2023x; 1.2023x over previous
import jax
import jax.numpy as jnp
from jax.experimental import pallas as pl

_MAX_STEPS = 4
_NUM_CLASSES = 2


def _mp_kernel(s_ref, x_ref, lab_ref, d_ref, ta_ref, tb_ref,
               recon_ref, pred_ref, sq_ref, ce_ref):
    i = pl.program_id(0)
    x = x_ref[...]
    D = d_ref[...]
    tA = ta_ref[...]
    tB = tb_ref[...]
    s = s_ref[0, 0]
    bt = x.shape[0]
    feat = x.shape[1]
    na = D.shape[1]

    residual = x
    recon = jnp.zeros_like(x)
    pred = jnp.zeros((bt, _NUM_CLASSES), jnp.float32)
    iota = jax.lax.broadcasted_iota(jnp.int32, (bt, na), 1)

    for t in range(_MAX_STEPS):
        proj = jnp.dot(residual, D, preferred_element_type=jnp.float32)
        absp = jnp.abs(proj)
        m = jnp.max(absp, axis=1, keepdims=True)
        best = jnp.min(jnp.where(absp == m, iota, na), axis=1, keepdims=True)
        hit = iota == best
        val = jnp.sum(jnp.where(hit, proj, 0.0), axis=1, keepdims=True)
        val = jnp.where(t < s, val, 0.0)
        dn = (((1,), (0,)), ((), ()))
        oh = hit.astype(jnp.bfloat16)
        dA = jax.lax.dot_general(oh, tA, dn,
                                 preferred_element_type=jnp.float32)
        dB = jax.lax.dot_general(oh, tB, dn,
                                 preferred_element_type=jnp.float32)
        atom = (dA[:, :feat] + dA[:, feat:2 * feat]) + dB[:, :feat]
        c01 = dB[:, feat:feat + 2 * _NUM_CLASSES]
        cvec = ((c01[:, :_NUM_CLASSES] + c01[:, _NUM_CLASSES:])
                + dB[:, feat + 2 * _NUM_CLASSES:feat + 3 * _NUM_CLASSES])
        upd = val * atom
        residual = residual - upd
        recon = recon + upd
        pred = pred + val * cvec

    recon_ref[...] = recon
    pred_ref[...] = pred

    lab = lab_ref[...]
    sq = jnp.sum((recon - x) ** 2).reshape(1, 1)
    mx = jnp.max(pred, axis=1, keepdims=True)
    lse = mx + jnp.log(jnp.sum(jnp.exp(pred - mx), axis=1, keepdims=True))
    ce = jnp.sum(lse[:, 0] - jnp.sum(pred * lab, axis=1)).reshape(1, 1)

    prev_sq = jnp.where(i == 0, jnp.zeros_like(sq), sq_ref[...])
    prev_ce = jnp.where(i == 0, jnp.zeros_like(ce), ce_ref[...])
    sq_ref[...] = prev_sq + sq
    ce_ref[...] = prev_ce + ce


def kernel(X, sparsity, labels, dictionary, classifier, beta):
    B, F = X.shape
    A = dictionary.shape[1]
    bt = 128
    nt = B // bt

    s = jnp.asarray(sparsity, jnp.int32).reshape(1, 1)
    lab1h = (labels[:, None] ==
             jnp.arange(_NUM_CLASSES, dtype=labels.dtype)[None, :]
             ).astype(jnp.float32)
    def split3(x):
        p0 = jax.lax.reduce_precision(x, 8, 7)
        r = x - p0
        p1 = jax.lax.reduce_precision(r, 8, 7)
        p2 = r - p1
        return (p0.astype(jnp.bfloat16), p1.astype(jnp.bfloat16),
                p2.astype(jnp.bfloat16))

    a0, a1, a2 = split3(dictionary.T)
    c0, c1, c2 = split3(classifier.T.astype(jnp.float32))
    tA = jnp.concatenate([a0, a1], axis=1)
    tB = jnp.concatenate([a2, c0, c1, c2], axis=1)

    recon, pred, sq, ce = pl.pallas_call(
        _mp_kernel,
        grid=(nt,),
        in_specs=[
            pl.BlockSpec((1, 1), lambda i: (0, 0)),
            pl.BlockSpec((bt, F), lambda i: (i, 0)),
            pl.BlockSpec((bt, _NUM_CLASSES), lambda i: (i, 0)),
            pl.BlockSpec((F, A), lambda i: (0, 0)),
            pl.BlockSpec((A, 2 * F), lambda i: (0, 0)),
            pl.BlockSpec((A, F + 3 * _NUM_CLASSES), lambda i: (0, 0)),
        ],
        out_specs=[
            pl.BlockSpec((bt, F), lambda i: (i, 0)),
            pl.BlockSpec((bt, _NUM_CLASSES), lambda i: (i, 0)),
            pl.BlockSpec((1, 1), lambda i: (0, 0)),
            pl.BlockSpec((1, 1), lambda i: (0, 0)),
        ],
        out_shape=[
            jax.ShapeDtypeStruct((B, F), jnp.float32),
            jax.ShapeDtypeStruct((B, _NUM_CLASSES), jnp.float32),
            jax.ShapeDtypeStruct((1, 1), jnp.float32),
            jax.ShapeDtypeStruct((1, 1), jnp.float32),
        ],
    )(s, X, lab1h, dictionary, tA, tB)

    mse = sq[0, 0] / (B * F)
    ce_mean = ce[0, 0] / B
    loss = (mse + beta * ce_mean).astype(jnp.float32)
    return recon, pred, loss


# speedup vs baseline: 1.5876x; 1.3205x over previous
import jax
import jax.numpy as jnp
from jax.experimental import pallas as pl

_MAX_STEPS = 4
_NUM_CLASSES = 2


def _mp_kernel(s_ref, x_ref, lab_ref, d_ref, ta_ref, tb_ref,
               recon_ref, pred_ref, sq_ref, ce_ref):
    i = pl.program_id(0)
    x = x_ref[...]
    D = d_ref[...]
    tA = ta_ref[...]
    tB = tb_ref[...]
    s = s_ref[0, 0]
    bt = x.shape[0]
    feat = x.shape[1]
    na = D.shape[1]

    residual = x
    recon = jnp.zeros_like(x)
    pred = jnp.zeros((bt, _NUM_CLASSES), jnp.float32)
    iota = jax.lax.broadcasted_iota(jnp.int32, (bt, na), 1)

    for t in range(_MAX_STEPS):
        proj = jnp.dot(residual, D, preferred_element_type=jnp.float32)
        absp = jnp.abs(proj)
        best = jnp.argmax(absp, axis=1, keepdims=True).astype(jnp.int32)
        hit = iota == best
        val = jnp.sum(jnp.where(hit, proj, 0.0), axis=1, keepdims=True)
        val = jnp.where(t < s, val, 0.0)
        dn = (((1,), (0,)), ((), ()))
        oh = hit.astype(jnp.bfloat16)
        dA = jax.lax.dot_general(oh, tA, dn,
                                 preferred_element_type=jnp.float32)
        dB = jax.lax.dot_general(oh, tB, dn,
                                 preferred_element_type=jnp.float32)
        atom = (dA[:, :feat] + dA[:, feat:2 * feat]) + dB[:, :feat]
        c01 = dB[:, feat:feat + 2 * _NUM_CLASSES]
        cvec = ((c01[:, :_NUM_CLASSES] + c01[:, _NUM_CLASSES:])
                + dB[:, feat + 2 * _NUM_CLASSES:feat + 3 * _NUM_CLASSES])
        upd = val * atom
        residual = residual - upd
        recon = recon + upd
        pred = pred + val * cvec

    recon_ref[...] = recon
    pred_ref[...] = pred

    lab = lab_ref[...]
    sq = jnp.sum((recon - x) ** 2).reshape(1, 1)
    mx = jnp.max(pred, axis=1, keepdims=True)
    lse = mx + jnp.log(jnp.sum(jnp.exp(pred - mx), axis=1, keepdims=True))
    ce = jnp.sum(lse[:, 0] - jnp.sum(pred * lab, axis=1)).reshape(1, 1)

    prev_sq = jnp.where(i == 0, jnp.zeros_like(sq), sq_ref[...])
    prev_ce = jnp.where(i == 0, jnp.zeros_like(ce), ce_ref[...])
    sq_ref[...] = prev_sq + sq
    ce_ref[...] = prev_ce + ce


def kernel(X, sparsity, labels, dictionary, classifier, beta):
    B, F = X.shape
    A = dictionary.shape[1]
    bt = 256
    nt = B // bt

    s = jnp.asarray(sparsity, jnp.int32).reshape(1, 1)
    lab1h = (labels[:, None] ==
             jnp.arange(_NUM_CLASSES, dtype=labels.dtype)[None, :]
             ).astype(jnp.float32)
    def split3(x):
        p0 = jax.lax.reduce_precision(x, 8, 7)
        r = x - p0
        p1 = jax.lax.reduce_precision(r, 8, 7)
        p2 = r - p1
        return (p0.astype(jnp.bfloat16), p1.astype(jnp.bfloat16),
                p2.astype(jnp.bfloat16))

    a0, a1, a2 = split3(dictionary.T)
    c0, c1, c2 = split3(classifier.T.astype(jnp.float32))
    tA = jnp.concatenate([a0, a1], axis=1)
    tB = jnp.concatenate([a2, c0, c1, c2], axis=1)

    recon, pred, sq, ce = pl.pallas_call(
        _mp_kernel,
        grid=(nt,),
        in_specs=[
            pl.BlockSpec((1, 1), lambda i: (0, 0)),
            pl.BlockSpec((bt, F), lambda i: (i, 0)),
            pl.BlockSpec((bt, _NUM_CLASSES), lambda i: (i, 0)),
            pl.BlockSpec((F, A), lambda i: (0, 0)),
            pl.BlockSpec((A, 2 * F), lambda i: (0, 0)),
            pl.BlockSpec((A, F + 3 * _NUM_CLASSES), lambda i: (0, 0)),
        ],
        out_specs=[
            pl.BlockSpec((bt, F), lambda i: (i, 0)),
            pl.BlockSpec((bt, _NUM_CLASSES), lambda i: (i, 0)),
            pl.BlockSpec((1, 1), lambda i: (0, 0)),
            pl.BlockSpec((1, 1), lambda i: (0, 0)),
        ],
        out_shape=[
            jax.ShapeDtypeStruct((B, F), jnp.float32),
            jax.ShapeDtypeStruct((B, _NUM_CLASSES), jnp.float32),
            jax.ShapeDtypeStruct((1, 1), jnp.float32),
            jax.ShapeDtypeStruct((1, 1), jnp.float32),
        ],
    )(s, X, lab1h, dictionary, tA, tB)

    mse = sq[0, 0] / (B * F)
    ce_mean = ce[0, 0] / B
    loss = (mse + beta * ce_mean).astype(jnp.float32)
    return recon, pred, loss
